# baseline (speedup 1.0000x reference)
import jax
import jax.numpy as jnp
from jax import lax
from jax.experimental import pallas as pl
from jax.experimental.pallas import tpu as pltpu
from jax.experimental.pallas import tpu_sc as plsc

NC = 2
NS = 16
NW = NC * NS
L = 16

B = 16384
E = 16
B_PER_W = B // NW
CHUNK = 128
NCHUNK = B_PER_W // CHUNK
GROUPS_PER_CHUNK = CHUNK // L

_BITREV = [int(f"{k:04b}"[::-1], 2) for k in range(L)]


def _gmf_body(user_hbm, item_hbm, uemb_hbm, iemb_hbm, out_hbm,
              uidx, iidx, urows, irows, outv, *sems):
    wid = lax.axis_index("s") * NC + lax.axis_index("c")
    base = wid * B_PER_W

    pltpu.sync_copy(user_hbm.at[wid], uidx)
    pltpu.sync_copy(item_hbm.at[wid], iidx)

    copies = []
    for j in range(NCHUNK):
        cu = pltpu.async_copy(uemb_hbm.at[uidx.at[j]],
                              urows.at[pl.ds(j * CHUNK, CHUNK)], sems[2 * j])
        ci = pltpu.async_copy(iemb_hbm.at[iidx.at[j]],
                              irows.at[pl.ds(j * CHUNK, CHUNK)], sems[2 * j + 1])
        copies.append((cu, ci))

    lanes = lax.iota(jnp.int32, L)
    masks = [(lanes & d) == 0 for d in (8, 4, 2, 1)]
    perms = [lanes ^ d for d in (8, 4, 2, 1)]

    def group(g, carry):
        rbase = g * L
        prods = [urows[rbase + _BITREV[k]] * irows[rbase + _BITREV[k]]
                 for k in range(L)]
        for lvl in range(4):
            m, p = masks[lvl], perms[lvl]
            nxt = []
            for a, b in zip(prods[0::2], prods[1::2]):
                t = jnp.where(m, a, b)
                u = jnp.where(m, b, a)
                nxt.append(t + jnp.take(u, p))
            prods = nxt
        dot = prods[0]
        outv[pl.ds(rbase, L)] = 1.0 / (1.0 + jnp.exp(-dot))
        return carry

    for j in range(NCHUNK):
        cu, ci = copies[j]
        cu.wait()
        ci.wait()
        lax.fori_loop(j * GROUPS_PER_CHUNK, (j + 1) * GROUPS_PER_CHUNK,
                      group, 0)

    pltpu.sync_copy(outv, out_hbm.at[pl.ds(base, B_PER_W)])


@jax.jit
def _gmf(user, item, user_embedding, item_embedding):
    mesh = plsc.VectorSubcoreMesh(
        core_axis_name="c", subcore_axis_name="s",
        num_cores=NC, num_subcores=NS)
    out = pl.kernel(
        _gmf_body,
        out_type=jax.ShapeDtypeStruct((B,), jnp.float32),
        mesh=mesh,
        scratch_types=[
            pltpu.VMEM((NCHUNK, CHUNK), jnp.int32),
            pltpu.VMEM((NCHUNK, CHUNK), jnp.int32),
            pltpu.VMEM((B_PER_W, E), jnp.float32),
            pltpu.VMEM((B_PER_W, E), jnp.float32),
            pltpu.VMEM((B_PER_W,), jnp.float32),
        ] + [pltpu.SemaphoreType.DMA] * (2 * NCHUNK),
        compiler_params=pltpu.CompilerParams(use_tc_tiling_on_sc=False),
    )(user, item, user_embedding, item_embedding)
    return out


def kernel(user, item, user_embedding, item_embedding):
    u = user.astype(jnp.int32).reshape(NW, NCHUNK, CHUNK)
    i = item.astype(jnp.int32).reshape(NW, NCHUNK, CHUNK)
    out = _gmf(u, i, user_embedding, item_embedding)
    return out.reshape(B, 1)
